# baseline (speedup 1.0000x reference)
import jax
import jax.numpy as jnp
from jax.experimental import pallas as pl

_C = 5
_K = 4
_W = 256
_BB = 2048
_DEF = jax.lax.Precision.DEFAULT


def _fused_kernel(dt_ref, pert_ref, w01_ref, w0_ref, w1_ref, h_ref, wl_ref,
                  out_ref):
    f32 = jnp.float32
    H = h_ref[...]
    wL = wl_ref[...]
    DT = dt_ref[...]

    UT = jnp.concatenate(
        [DT + pert_ref[:, c:c + 1] for c in range(_C)], axis=1)
    nrm = jnp.sqrt(jnp.sum(UT * UT, axis=0, keepdims=True))
    XT = UT / nrm

    zz = jax.lax.dot_general(w01_ref[...], XT, (((0,), (0,)), ((), ())),
                             preferred_element_type=f32, precision=_DEF)
    z1 = zz[0:_W, :]
    h1 = jnp.maximum(z1, 0.0)
    z2 = jax.lax.dot_general(H, h1, (((0,), (0,)), ((), ())),
                             preferred_element_type=f32,
                             precision=_DEF) + zz[_W:2 * _W, :]
    g2 = jnp.where(z2 > 0.0, wL, 0.0)
    t = jax.lax.dot_general(H, g2, (((1,), (0,)), ((), ())),
                            preferred_element_type=f32, precision=_DEF)
    g1 = jnp.where(z1 > 0.0, t, 0.0)
    v = (jax.lax.dot_general(w1_ref[...], g2, (((1,), (0,)), ((), ())),
                             preferred_element_type=f32, precision=_DEF)
         + jax.lax.dot_general(w0_ref[...], g1, (((1,), (0,)), ((), ())),
                               preferred_element_type=f32,
                               precision=_DEF))

    d0 = DT[0:1, :]
    d1 = DT[1:2, :]
    d2 = DT[2:3, :]
    verts = []
    dots = []
    for c in range(_C):
        vc = v[:, c * _BB:(c + 1) * _BB]
        verts.append(vc)
        dots.append(d0 * vc[0:1, :] + d1 * vc[1:2, :]
                    + d2 * vc[2:3, :])

    ranks = []
    for c in range(_C):
        r = jnp.zeros_like(dots[c], dtype=jnp.int32)
        for j in range(_C):
            if j == c:
                continue
            beats = dots[j] > dots[c]
            if j < c:
                beats = beats | (dots[j] == dots[c])
            r = r + beats.astype(jnp.int32)
        ranks.append(r)

    for r in range(_K):
        o = jnp.zeros_like(verts[0])
        for c in range(_C):
            sel = (ranks[c] == r).astype(jnp.float32)
            o = o + sel * verts[c]
        out_ref[3 * r:3 * r + 3, :] = o


@jax.jit
def _run(directions_t, pert_t, W01, W_in0, W_in1, H, wL):
    B = directions_t.shape[1]
    grid = (B // _BB,)
    out = pl.pallas_call(
        _fused_kernel,
        grid=grid,
        in_specs=[
            pl.BlockSpec((3, _BB), lambda i: (0, i)),
            pl.BlockSpec((3, _C), lambda i: (0, 0)),
            pl.BlockSpec((3, 2 * _W), lambda i: (0, 0)),
            pl.BlockSpec((3, _W), lambda i: (0, 0)),
            pl.BlockSpec((3, _W), lambda i: (0, 0)),
            pl.BlockSpec((_W, _W), lambda i: (0, 0)),
            pl.BlockSpec((_W, 1), lambda i: (0, 0)),
        ],
        out_specs=pl.BlockSpec((3 * _K, _BB), lambda i: (0, i)),
        out_shape=jax.ShapeDtypeStruct((3 * _K, B), jnp.float32),
    )(directions_t, pert_t, W01, W_in0, W_in1, H, wL)
    return out.T.reshape(B, _K, 3)


def kernel(directions, perturbations, W_in0, W_in1, W_hid0_log, w_out_log,
           length_scale):
    pert_full = jnp.concatenate(
        [jnp.zeros((1, 3), directions.dtype), perturbations], axis=0)
    H = jnp.exp(W_hid0_log)
    wL = (jnp.exp(w_out_log) * length_scale).reshape(_W, 1)
    W01 = jnp.concatenate([W_in0, W_in1], axis=1)
    return _run(directions.T, pert_full.T, W01, W_in0, W_in1, H, wL)
